# baseline (speedup 1.0000x reference)
import functools

import jax
import jax.numpy as jnp
from jax import lax
from jax.experimental import pallas as pl
from jax.experimental.pallas import tpu as pltpu
from jax.experimental.pallas import tpu_sc as plsc

D = 16
NW = 32
CH = 128


def _mesh():
    return plsc.VectorSubcoreMesh(core_axis_name="c", subcore_axis_name="s")


NBUF = 8


def _seg_sum_body(n_chunks, rows_per_sub, tbl_hbm, src_hbm, dst_hbm, zeros_hbm,
                  out_hbm, src_v, dst_v, bufs, gsem, ssem, acc, tbl_s):
    c = lax.axis_index("c")
    s = lax.axis_index("s")
    wid = s * 2 + c
    sl = pl.ds(s * rows_per_sub, rows_per_sub)
    pltpu.sync_copy(zeros_hbm.at[sl], acc.at[sl])
    pltpu.sync_copy(tbl_hbm.at[sl], tbl_s.at[sl])
    pltpu.sync_copy(src_hbm.at[wid], src_v)
    pltpu.sync_copy(dst_hbm.at[wid], dst_v)
    plsc.subcore_barrier()

    def gather_start(j, b):
        pltpu.async_copy(tbl_s.at[src_v.at[j]], bufs.at[b], gsem.at[b])

    def gather_wait(b):
        pltpu.make_async_copy(zeros_hbm.at[pl.ds(0, CH)], bufs.at[b],
                              gsem.at[b]).wait()

    def scatter_start(j, b):
        pltpu.async_copy(bufs.at[b], acc.at[dst_v.at[j]], ssem.at[b], add=True)

    def scatter_wait(b):
        pltpu.make_async_copy(bufs.at[b], acc.at[pl.ds(0, CH)],
                              ssem.at[b]).wait()

    for b in range(NBUF):
        gather_start(b, b)

    n_outer = n_chunks // NBUF

    def outer(o, carry):
        for b in range(NBUF):
            gather_wait(b)
            scatter_start(o * NBUF + b, b)

        @pl.when(o < n_outer - 1)
        def _prefetch():
            for b in range(NBUF):
                scatter_wait(b)
                gather_start((o + 1) * NBUF + b, b)

        return carry

    lax.fori_loop(0, n_outer, outer, 0)
    for b in range(NBUF):
        scatter_wait(b)
    plsc.subcore_barrier()
    pltpu.sync_copy(acc.at[sl], out_hbm.at[c, sl])


def _seg_sum2_body(n_chunks, rows_per_sub, s1p_hbm, self1_hbm, dis_hbm,
                   b1_hbm, src_hbm, dst_hbm, zeros_hbm,
                   out_hbm, self2_hbm, src_v, dst_v, bufs, gsem, ssem,
                   p0_v, p1_v, self1_v, dis_v, g2_v, self2_v, b1_v,
                   acc, tbl_s):
    c = lax.axis_index("c")
    s = lax.axis_index("s")
    wid = s * 2 + c
    base = s * rows_per_sub
    sl = pl.ds(base, rows_per_sub)
    pltpu.sync_copy(zeros_hbm.at[sl], acc.at[sl])
    pltpu.sync_copy(s1p_hbm.at[0, sl], p0_v)
    pltpu.sync_copy(s1p_hbm.at[1, sl], p1_v)
    pltpu.sync_copy(self1_hbm.at[sl], self1_v)
    pltpu.sync_copy(dis_hbm.at[sl], dis_v)
    pltpu.sync_copy(b1_hbm, b1_v)
    pltpu.sync_copy(src_hbm.at[wid], src_v)
    pltpu.sync_copy(dst_hbm.at[wid], dst_v)
    b1 = b1_v[...]

    def row(r, carry):
        dis = dis_v[r]
        h = jnp.maximum(dis * (p0_v[r] + p1_v[r]) + self1_v[r] + b1, 0.0)
        g2_v[r] = h * dis
        self2_v[r] = h * dis * dis
        return carry

    lax.fori_loop(0, rows_per_sub, row, 0)
    pltpu.sync_copy(g2_v, tbl_s.at[sl])

    @pl.when(c == 0)
    def _write_self2():
        pltpu.sync_copy(self2_v, self2_hbm.at[sl])

    plsc.subcore_barrier()

    def gather_start(j, b):
        pltpu.async_copy(tbl_s.at[src_v.at[j]], bufs.at[b], gsem.at[b])

    def gather_wait(b):
        pltpu.make_async_copy(zeros_hbm.at[pl.ds(0, CH)], bufs.at[b],
                              gsem.at[b]).wait()

    def scatter_start(j, b):
        pltpu.async_copy(bufs.at[b], acc.at[dst_v.at[j]], ssem.at[b], add=True)

    def scatter_wait(b):
        pltpu.make_async_copy(bufs.at[b], acc.at[pl.ds(0, CH)],
                              ssem.at[b]).wait()

    for b in range(NBUF):
        gather_start(b, b)

    n_outer = n_chunks // NBUF

    def outer(o, carry):
        for b in range(NBUF):
            gather_wait(b)
            scatter_start(o * NBUF + b, b)

        @pl.when(o < n_outer - 1)
        def _prefetch():
            for b in range(NBUF):
                scatter_wait(b)
                gather_start((o + 1) * NBUF + b, b)

        return carry

    lax.fori_loop(0, n_outer, outer, 0)
    for b in range(NBUF):
        scatter_wait(b)
    plsc.subcore_barrier()
    pltpu.sync_copy(acc.at[sl], out_hbm.at[c, sl])


def _seg_sum2_call(n_chunks, npad, s1p, self1, dis, b1, src, dst, zeros):
    rps = npad // 16
    f = pl.kernel(
        functools.partial(_seg_sum2_body, n_chunks, rps),
        out_type=(
            jax.ShapeDtypeStruct((2, npad, D), jnp.float32),
            jax.ShapeDtypeStruct((npad, D), jnp.float32),
        ),
        mesh=_mesh(),
        scratch_types=[
            pltpu.VMEM((n_chunks, CH), jnp.int32),
            pltpu.VMEM((n_chunks, CH), jnp.int32),
            pltpu.VMEM((NBUF, CH, D), jnp.float32),
            pltpu.SemaphoreType.DMA((NBUF,)),
            pltpu.SemaphoreType.DMA((NBUF,)),
            pltpu.VMEM((rps, D), jnp.float32),
            pltpu.VMEM((rps, D), jnp.float32),
            pltpu.VMEM((rps, D), jnp.float32),
            pltpu.VMEM((rps, D), jnp.float32),
            pltpu.VMEM((rps, D), jnp.float32),
            pltpu.VMEM((rps, D), jnp.float32),
            pltpu.VMEM((D,), jnp.float32),
            pltpu.VMEM_SHARED((npad, D), jnp.float32),
            pltpu.VMEM_SHARED((npad, D), jnp.float32),
        ],
        compiler_params=pltpu.CompilerParams(use_tc_tiling_on_sc=False, skip_device_barrier=True),
    )
    return f(s1p, self1, dis, b1, src, dst, zeros)


def _deg_body(n_chunks, rows_per_sub, dst_hbm, ones_hbm, zeros_hbm,
              out_hbm, dst_v, ones_v, sem, acc):
    c = lax.axis_index("c")
    s = lax.axis_index("s")
    wid = s * 2 + c
    sl = pl.ds(s * rows_per_sub, rows_per_sub)
    pltpu.sync_copy(zeros_hbm.at[sl], acc.at[sl])
    pltpu.sync_copy(ones_hbm, ones_v)
    pltpu.sync_copy(dst_hbm.at[wid], dst_v)
    plsc.subcore_barrier()

    n_outer = n_chunks // NBUF

    def body(o, carry):
        for b in range(NBUF):
            pltpu.async_copy(ones_v, acc.at[dst_v.at[o * NBUF + b]], sem,
                             add=True)
        for b in range(NBUF):
            pltpu.make_async_copy(ones_v, acc.at[pl.ds(0, CH)], sem).wait()
        return carry

    lax.fori_loop(0, n_outer, body, 0)
    plsc.subcore_barrier()
    pltpu.sync_copy(acc.at[sl], out_hbm.at[c, sl])


def _seg_sum_call(n_chunks, npad, tbl, src, dst, zeros):
    rps = npad // 16
    f = pl.kernel(
        functools.partial(_seg_sum_body, n_chunks, rps),
        out_type=jax.ShapeDtypeStruct((2, npad, D), jnp.float32),
        mesh=_mesh(),
        scratch_types=[
            pltpu.VMEM((n_chunks, CH), jnp.int32),
            pltpu.VMEM((n_chunks, CH), jnp.int32),
            pltpu.VMEM((NBUF, CH, D), jnp.float32),
            pltpu.SemaphoreType.DMA((NBUF,)),
            pltpu.SemaphoreType.DMA((NBUF,)),
            pltpu.VMEM_SHARED((npad, D), jnp.float32),
            pltpu.VMEM_SHARED((npad, D), jnp.float32),
        ],
        compiler_params=pltpu.CompilerParams(use_tc_tiling_on_sc=False, skip_device_barrier=True),
    )
    return f(tbl, src, dst, zeros)


def _deg_call(n_chunks, npad, dst, ones1, zeros1):
    rps = npad // 16
    f = pl.kernel(
        functools.partial(_deg_body, n_chunks, rps),
        out_type=jax.ShapeDtypeStruct((2, npad), jnp.float32),
        mesh=_mesh(),
        scratch_types=[
            pltpu.VMEM((n_chunks, CH), jnp.int32),
            pltpu.VMEM((CH,), jnp.float32),
            pltpu.SemaphoreType.DMA,
            pltpu.VMEM_SHARED((npad,), jnp.float32),
        ],
        compiler_params=pltpu.CompilerParams(use_tc_tiling_on_sc=False, skip_device_barrier=True),
    )
    return f(dst, ones1, zeros1)


def _tc_b_body(x_ref, w1_ref, degp_ref, g1_ref, self1_ref, dis_ref):
    deg = degp_ref[0] + degp_ref[1] + 1.0
    dis = lax.rsqrt(deg)
    h1 = jnp.dot(x_ref[...], w1_ref[...], preferred_element_type=jnp.float32)
    g1_ref[...] = h1 * dis
    self1_ref[...] = h1 / deg
    dis_ref[...] = jnp.broadcast_to(dis, h1.shape)


def _tc_f_body(s2p_ref, self2_ref, dis_ref, w2_ref, b2_ref, out_ref):
    a2 = dis_ref[...] * (s2p_ref[0] + s2p_ref[1]) + self2_ref[...]
    out_ref[...] = (
        jnp.dot(a2, w2_ref[...], preferred_element_type=jnp.float32) + b2_ref[...]
    )


def kernel(x, edge_index, W1, b1, W2, b2):
    n, d_in = x.shape
    e = edge_index.shape[1]
    d_out = W2.shape[1]
    n_chunks = -(-e // (NW * CH * NBUF)) * NBUF
    epad = NW * n_chunks * CH
    npad = -(-(n + 1) // 128) * 128

    idt = edge_index.dtype
    pad_idx = jnp.full((epad - e,), n, dtype=idt)
    src = jnp.concatenate([edge_index[0], pad_idx]).reshape(NW, n_chunks, CH)
    dst = jnp.concatenate([edge_index[1], pad_idx]).reshape(NW, n_chunks, CH)
    zeros = jnp.zeros((npad, D), jnp.float32)
    ones1 = jnp.ones((CH,), jnp.float32)
    zeros1 = jnp.zeros((npad,), jnp.float32)

    degp = _deg_call(n_chunks, npad, dst, ones1, zeros1)

    blk = 2000
    grid = n // blk
    g1, self1, dis = pl.pallas_call(
        _tc_b_body,
        grid=(grid,),
        in_specs=[
            pl.BlockSpec((blk, d_in), lambda i: (i, 0)),
            pl.BlockSpec((d_in, D), lambda i: (0, 0)),
            pl.BlockSpec((2, blk, 1), lambda i: (0, i, 0)),
        ],
        out_specs=[pl.BlockSpec((blk, D), lambda i: (i, 0))] * 3,
        out_shape=[jax.ShapeDtypeStruct((npad, D), jnp.float32)] * 3,
    )(x, W1, degp[:, :, None])

    s1p = _seg_sum_call(n_chunks, npad, g1, src, dst, zeros)

    s2p, self2 = _seg_sum2_call(n_chunks, npad, s1p, self1, dis, b1, src,
                                dst, zeros)

    out = pl.pallas_call(
        _tc_f_body,
        grid=(grid,),
        in_specs=[
            pl.BlockSpec((2, blk, D), lambda i: (0, i, 0)),
            pl.BlockSpec((blk, D), lambda i: (i, 0)),
            pl.BlockSpec((blk, D), lambda i: (i, 0)),
            pl.BlockSpec((D, d_out), lambda i: (0, 0)),
            pl.BlockSpec((1, d_out), lambda i: (0, 0)),
        ],
        out_specs=pl.BlockSpec((blk, d_out), lambda i: (i, 0)),
        out_shape=jax.ShapeDtypeStruct((n, d_out), jnp.float32),
    )(s2p, self2, dis, W2, b2.reshape(1, d_out))

    return out


# speedup vs baseline: 1.0017x; 1.0017x over previous
import functools

import jax
import jax.numpy as jnp
from jax import lax
from jax.experimental import pallas as pl
from jax.experimental.pallas import tpu as pltpu
from jax.experimental.pallas import tpu_sc as plsc

D = 16
NW = 32
CH = 128


def _mesh():
    return plsc.VectorSubcoreMesh(core_axis_name="c", subcore_axis_name="s")


NBUF = 8


def _seg_sum_body(n_chunks, rows_per_sub, tbl_hbm, src_hbm, dst_hbm, zeros_hbm,
                  out_hbm, src_v, dst_v, bufs, gsem, ssem, acc, tbl_s):
    c = lax.axis_index("c")
    s = lax.axis_index("s")
    wid = s * 2 + c
    sl = pl.ds(s * rows_per_sub, rows_per_sub)
    pltpu.sync_copy(zeros_hbm.at[sl], acc.at[sl])
    pltpu.sync_copy(tbl_hbm.at[sl], tbl_s.at[sl])
    pltpu.sync_copy(src_hbm.at[wid], src_v)
    pltpu.sync_copy(dst_hbm.at[wid], dst_v)
    plsc.subcore_barrier()

    def gather_start(j, b):
        pltpu.async_copy(tbl_s.at[src_v.at[j]], bufs.at[b], gsem.at[b])

    def gather_wait(b):
        pltpu.make_async_copy(zeros_hbm.at[pl.ds(0, CH)], bufs.at[b],
                              gsem.at[b]).wait()

    def scatter_start(j, b):
        pltpu.async_copy(bufs.at[b], acc.at[dst_v.at[j]], ssem.at[b], add=True)

    def scatter_wait(b):
        pltpu.make_async_copy(bufs.at[b], acc.at[pl.ds(0, CH)],
                              ssem.at[b]).wait()

    for b in range(NBUF):
        gather_start(b, b)

    n_outer = n_chunks // NBUF

    def outer(o, carry):
        for b in range(NBUF):
            gather_wait(b)
            scatter_start(o * NBUF + b, b)

        @pl.when(o < n_outer - 1)
        def _prefetch():
            for b in range(NBUF):
                scatter_wait(b)
                gather_start((o + 1) * NBUF + b, b)

        return carry

    lax.fori_loop(0, n_outer, outer, 0)
    for b in range(NBUF):
        scatter_wait(b)
    plsc.subcore_barrier()
    pltpu.sync_copy(acc.at[sl], out_hbm.at[c, sl])


def _seg_sum2_body(n_chunks, rows_per_sub, s1p_hbm, self1_hbm, dis_hbm,
                   b1_hbm, src_hbm, dst_hbm, zeros_hbm,
                   out_hbm, self2_hbm, src_v, dst_v, bufs, gsem, ssem,
                   p0_v, p1_v, self1_v, dis_v, g2_v, self2_v, b1_v,
                   acc, tbl_s):
    c = lax.axis_index("c")
    s = lax.axis_index("s")
    wid = s * 2 + c
    base = s * rows_per_sub
    sl = pl.ds(base, rows_per_sub)
    pltpu.sync_copy(zeros_hbm.at[sl], acc.at[sl])
    pltpu.sync_copy(s1p_hbm.at[0, sl], p0_v)
    pltpu.sync_copy(s1p_hbm.at[1, sl], p1_v)
    pltpu.sync_copy(self1_hbm.at[sl], self1_v)
    pltpu.sync_copy(dis_hbm.at[sl], dis_v)
    pltpu.sync_copy(b1_hbm, b1_v)
    pltpu.sync_copy(src_hbm.at[wid], src_v)
    pltpu.sync_copy(dst_hbm.at[wid], dst_v)
    b1 = b1_v[...]

    def row(r, carry):
        dis = dis_v[r]
        h = jnp.maximum(dis * (p0_v[r] + p1_v[r]) + self1_v[r] + b1, 0.0)
        g2_v[r] = h * dis
        self2_v[r] = h * dis * dis
        return carry

    lax.fori_loop(0, rows_per_sub, row, 0)
    pltpu.sync_copy(g2_v, tbl_s.at[sl])

    @pl.when(c == 0)
    def _write_self2():
        pltpu.sync_copy(self2_v, self2_hbm.at[sl])

    plsc.subcore_barrier()

    def gather_start(j, b):
        pltpu.async_copy(tbl_s.at[src_v.at[j]], bufs.at[b], gsem.at[b])

    def gather_wait(b):
        pltpu.make_async_copy(zeros_hbm.at[pl.ds(0, CH)], bufs.at[b],
                              gsem.at[b]).wait()

    def scatter_start(j, b):
        pltpu.async_copy(bufs.at[b], acc.at[dst_v.at[j]], ssem.at[b], add=True)

    def scatter_wait(b):
        pltpu.make_async_copy(bufs.at[b], acc.at[pl.ds(0, CH)],
                              ssem.at[b]).wait()

    for b in range(NBUF):
        gather_start(b, b)

    n_outer = n_chunks // NBUF

    def outer(o, carry):
        for b in range(NBUF):
            gather_wait(b)
            scatter_start(o * NBUF + b, b)

        @pl.when(o < n_outer - 1)
        def _prefetch():
            for b in range(NBUF):
                scatter_wait(b)
                gather_start((o + 1) * NBUF + b, b)

        return carry

    lax.fori_loop(0, n_outer, outer, 0)
    for b in range(NBUF):
        scatter_wait(b)
    plsc.subcore_barrier()
    pltpu.sync_copy(acc.at[sl], out_hbm.at[c, sl])


def _seg_sum2_call(n_chunks, npad, s1p, self1, dis, b1, src, dst, zeros):
    rps = npad // 16
    f = pl.kernel(
        functools.partial(_seg_sum2_body, n_chunks, rps),
        out_type=(
            jax.ShapeDtypeStruct((2, npad, D), jnp.float32),
            jax.ShapeDtypeStruct((npad, D), jnp.float32),
        ),
        mesh=_mesh(),
        scratch_types=[
            pltpu.VMEM((n_chunks, CH), jnp.int32),
            pltpu.VMEM((n_chunks, CH), jnp.int32),
            pltpu.VMEM((NBUF, CH, D), jnp.float32),
            pltpu.SemaphoreType.DMA((NBUF,)),
            pltpu.SemaphoreType.DMA((NBUF,)),
            pltpu.VMEM((rps, D), jnp.float32),
            pltpu.VMEM((rps, D), jnp.float32),
            pltpu.VMEM((rps, D), jnp.float32),
            pltpu.VMEM((rps, D), jnp.float32),
            pltpu.VMEM((rps, D), jnp.float32),
            pltpu.VMEM((rps, D), jnp.float32),
            pltpu.VMEM((D,), jnp.float32),
            pltpu.VMEM_SHARED((npad, D), jnp.float32),
            pltpu.VMEM_SHARED((npad, D), jnp.float32),
        ],
        compiler_params=pltpu.CompilerParams(use_tc_tiling_on_sc=False),
    )
    return f(s1p, self1, dis, b1, src, dst, zeros)


def _deg_body(n_chunks, rows_per_sub, dst_hbm, ones_hbm, zeros_hbm,
              out_hbm, dst_v, ones_v, sem, acc):
    c = lax.axis_index("c")
    s = lax.axis_index("s")
    wid = s * 2 + c
    sl = pl.ds(s * rows_per_sub, rows_per_sub)
    pltpu.sync_copy(zeros_hbm.at[sl], acc.at[sl])
    pltpu.sync_copy(ones_hbm, ones_v)
    pltpu.sync_copy(dst_hbm.at[wid], dst_v)
    plsc.subcore_barrier()

    n_outer = n_chunks // NBUF

    def body(o, carry):
        for b in range(NBUF):
            pltpu.async_copy(ones_v, acc.at[dst_v.at[o * NBUF + b]], sem,
                             add=True)
        for b in range(NBUF):
            pltpu.make_async_copy(ones_v, acc.at[pl.ds(0, CH)], sem).wait()
        return carry

    lax.fori_loop(0, n_outer, body, 0)
    plsc.subcore_barrier()
    pltpu.sync_copy(acc.at[sl], out_hbm.at[c, sl])


def _seg_sum_call(n_chunks, npad, tbl, src, dst, zeros):
    rps = npad // 16
    f = pl.kernel(
        functools.partial(_seg_sum_body, n_chunks, rps),
        out_type=jax.ShapeDtypeStruct((2, npad, D), jnp.float32),
        mesh=_mesh(),
        scratch_types=[
            pltpu.VMEM((n_chunks, CH), jnp.int32),
            pltpu.VMEM((n_chunks, CH), jnp.int32),
            pltpu.VMEM((NBUF, CH, D), jnp.float32),
            pltpu.SemaphoreType.DMA((NBUF,)),
            pltpu.SemaphoreType.DMA((NBUF,)),
            pltpu.VMEM_SHARED((npad, D), jnp.float32),
            pltpu.VMEM_SHARED((npad, D), jnp.float32),
        ],
        compiler_params=pltpu.CompilerParams(use_tc_tiling_on_sc=False),
    )
    return f(tbl, src, dst, zeros)


def _deg_call(n_chunks, npad, dst, ones1, zeros1):
    rps = npad // 16
    f = pl.kernel(
        functools.partial(_deg_body, n_chunks, rps),
        out_type=jax.ShapeDtypeStruct((2, npad), jnp.float32),
        mesh=_mesh(),
        scratch_types=[
            pltpu.VMEM((n_chunks, CH), jnp.int32),
            pltpu.VMEM((CH,), jnp.float32),
            pltpu.SemaphoreType.DMA,
            pltpu.VMEM_SHARED((npad,), jnp.float32),
        ],
        compiler_params=pltpu.CompilerParams(use_tc_tiling_on_sc=False),
    )
    return f(dst, ones1, zeros1)


def _tc_b_body(x_ref, w1_ref, degp_ref, g1_ref, self1_ref, dis_ref):
    deg = degp_ref[0] + degp_ref[1] + 1.0
    dis = lax.rsqrt(deg)
    h1 = jnp.dot(x_ref[...], w1_ref[...], preferred_element_type=jnp.float32)
    g1_ref[...] = h1 * dis
    self1_ref[...] = h1 / deg
    dis_ref[...] = jnp.broadcast_to(dis, h1.shape)


def _tc_f_body(s2p_ref, self2_ref, dis_ref, w2_ref, b2_ref, out_ref):
    a2 = dis_ref[...] * (s2p_ref[0] + s2p_ref[1]) + self2_ref[...]
    out_ref[...] = (
        jnp.dot(a2, w2_ref[...], preferred_element_type=jnp.float32) + b2_ref[...]
    )


def kernel(x, edge_index, W1, b1, W2, b2):
    n, d_in = x.shape
    e = edge_index.shape[1]
    d_out = W2.shape[1]
    n_chunks = -(-e // (NW * CH * NBUF)) * NBUF
    epad = NW * n_chunks * CH
    npad = -(-(n + 1) // 128) * 128

    idt = edge_index.dtype
    pad_idx = jnp.full((epad - e,), n, dtype=idt)
    src = jnp.concatenate([edge_index[0], pad_idx]).reshape(NW, n_chunks, CH)
    dst = jnp.concatenate([edge_index[1], pad_idx]).reshape(NW, n_chunks, CH)
    zeros = jnp.zeros((npad, D), jnp.float32)
    ones1 = jnp.ones((CH,), jnp.float32)
    zeros1 = jnp.zeros((npad,), jnp.float32)

    degp = _deg_call(n_chunks, npad, dst, ones1, zeros1)

    blk = 2000
    grid = n // blk
    g1, self1, dis = pl.pallas_call(
        _tc_b_body,
        grid=(grid,),
        in_specs=[
            pl.BlockSpec((blk, d_in), lambda i: (i, 0)),
            pl.BlockSpec((d_in, D), lambda i: (0, 0)),
            pl.BlockSpec((2, blk, 1), lambda i: (0, i, 0)),
        ],
        out_specs=[pl.BlockSpec((blk, D), lambda i: (i, 0))] * 3,
        out_shape=[jax.ShapeDtypeStruct((npad, D), jnp.float32)] * 3,
    )(x, W1, degp[:, :, None])

    s1p = _seg_sum_call(n_chunks, npad, g1, src, dst, zeros)

    s2p, self2 = _seg_sum2_call(n_chunks, npad, s1p, self1, dis, b1, src,
                                dst, zeros)

    out = pl.pallas_call(
        _tc_f_body,
        grid=(grid,),
        in_specs=[
            pl.BlockSpec((2, blk, D), lambda i: (0, i, 0)),
            pl.BlockSpec((blk, D), lambda i: (i, 0)),
            pl.BlockSpec((blk, D), lambda i: (i, 0)),
            pl.BlockSpec((D, d_out), lambda i: (0, 0)),
            pl.BlockSpec((1, d_out), lambda i: (0, 0)),
        ],
        out_specs=pl.BlockSpec((blk, d_out), lambda i: (i, 0)),
        out_shape=jax.ShapeDtypeStruct((n, d_out), jnp.float32),
    )(s2p, self2, dis, W2, b2.reshape(1, d_out))

    return out


# speedup vs baseline: 1.0029x; 1.0013x over previous
import functools

import jax
import jax.numpy as jnp
from jax import lax
from jax.experimental import pallas as pl
from jax.experimental.pallas import tpu as pltpu
from jax.experimental.pallas import tpu_sc as plsc

D = 16
NW = 32
CH = 128


def _mesh():
    return plsc.VectorSubcoreMesh(core_axis_name="c", subcore_axis_name="s")


NBUF = 8


def _seg_sum_body(n_chunks, rows_per_sub, tbl_hbm, src_hbm, dst_hbm, zeros_hbm,
                  out_hbm, src_v, dst_v, bufs, gsem, ssem, acc, tbl_s):
    c = lax.axis_index("c")
    s = lax.axis_index("s")
    wid = s * 2 + c
    sl = pl.ds(s * rows_per_sub, rows_per_sub)
    pltpu.sync_copy(zeros_hbm.at[sl], acc.at[sl])
    pltpu.sync_copy(tbl_hbm.at[sl], tbl_s.at[sl])
    pltpu.sync_copy(src_hbm.at[wid], src_v)
    pltpu.sync_copy(dst_hbm.at[wid], dst_v)
    plsc.subcore_barrier()

    def gather_start(j, b):
        pltpu.async_copy(tbl_s.at[src_v.at[j]], bufs.at[b], gsem.at[b])

    def gather_wait(b):
        pltpu.make_async_copy(zeros_hbm.at[pl.ds(0, CH)], bufs.at[b],
                              gsem.at[b]).wait()

    def scatter_start(j, b):
        pltpu.async_copy(bufs.at[b], acc.at[dst_v.at[j]], ssem.at[b], add=True)

    def scatter_wait(b):
        pltpu.make_async_copy(bufs.at[b], acc.at[pl.ds(0, CH)],
                              ssem.at[b]).wait()

    for b in range(NBUF):
        gather_start(b, b)

    n_outer = n_chunks // NBUF

    def outer(o, carry):
        for b in range(NBUF):
            gather_wait(b)
            scatter_start(o * NBUF + b, b)

        @pl.when(o < n_outer - 1)
        def _prefetch():
            for b in range(NBUF):
                scatter_wait(b)
                gather_start((o + 1) * NBUF + b, b)

        return carry

    lax.fori_loop(0, n_outer, outer, 0)
    for b in range(NBUF):
        scatter_wait(b)
    plsc.subcore_barrier()
    pltpu.sync_copy(acc.at[sl], out_hbm.at[c, sl])


def _seg_sum2_body(n_chunks, rows_per_sub, s1p_hbm, self1_hbm, dis_hbm,
                   b1_hbm, src_hbm, dst_hbm, zeros_hbm,
                   out_hbm, self2_hbm, src_v, dst_v, bufs, gsem, ssem,
                   p0_v, p1_v, self1_v, dis_v, g2_v, self2_v, b1_v,
                   acc, tbl_s):
    c = lax.axis_index("c")
    s = lax.axis_index("s")
    wid = s * 2 + c
    base = s * rows_per_sub
    sl = pl.ds(base, rows_per_sub)
    pltpu.sync_copy(zeros_hbm.at[sl], acc.at[sl])
    pltpu.sync_copy(s1p_hbm.at[0, sl], p0_v)
    pltpu.sync_copy(s1p_hbm.at[1, sl], p1_v)
    pltpu.sync_copy(self1_hbm.at[sl], self1_v)
    pltpu.sync_copy(dis_hbm.at[sl], dis_v)
    pltpu.sync_copy(b1_hbm, b1_v)
    pltpu.sync_copy(src_hbm.at[wid], src_v)
    pltpu.sync_copy(dst_hbm.at[wid], dst_v)
    b1 = b1_v[...]

    def row(r, carry):
        dis = dis_v[r]
        h = jnp.maximum(dis * (p0_v[r] + p1_v[r]) + self1_v[r] + b1, 0.0)
        g2_v[r] = h * dis
        self2_v[r] = h * dis * dis
        return carry

    lax.fori_loop(0, rows_per_sub, row, 0)
    pltpu.sync_copy(g2_v, tbl_s.at[sl])

    @pl.when(c == 0)
    def _write_self2():
        pltpu.sync_copy(self2_v, self2_hbm.at[sl])

    plsc.subcore_barrier()

    def gather_start(j, b):
        pltpu.async_copy(tbl_s.at[src_v.at[j]], bufs.at[b], gsem.at[b])

    def gather_wait(b):
        pltpu.make_async_copy(zeros_hbm.at[pl.ds(0, CH)], bufs.at[b],
                              gsem.at[b]).wait()

    def scatter_start(j, b):
        pltpu.async_copy(bufs.at[b], acc.at[dst_v.at[j]], ssem.at[b], add=True)

    def scatter_wait(b):
        pltpu.make_async_copy(bufs.at[b], acc.at[pl.ds(0, CH)],
                              ssem.at[b]).wait()

    for b in range(NBUF):
        gather_start(b, b)

    n_outer = n_chunks // NBUF

    def outer(o, carry):
        for b in range(NBUF):
            gather_wait(b)
            scatter_start(o * NBUF + b, b)

        @pl.when(o < n_outer - 1)
        def _prefetch():
            for b in range(NBUF):
                scatter_wait(b)
                gather_start((o + 1) * NBUF + b, b)

        return carry

    lax.fori_loop(0, n_outer, outer, 0)
    for b in range(NBUF):
        scatter_wait(b)
    plsc.subcore_barrier()
    pltpu.sync_copy(acc.at[sl], out_hbm.at[c, sl])


def _seg_sum2_call(n_chunks, npad, s1p, self1, dis, b1, src, dst, zeros):
    rps = npad // 16
    f = pl.kernel(
        functools.partial(_seg_sum2_body, n_chunks, rps),
        out_type=(
            jax.ShapeDtypeStruct((2, npad, D), jnp.float32),
            jax.ShapeDtypeStruct((npad, D), jnp.float32),
        ),
        mesh=_mesh(),
        scratch_types=[
            pltpu.VMEM((n_chunks, CH), jnp.int32),
            pltpu.VMEM((n_chunks, CH), jnp.int32),
            pltpu.VMEM((NBUF, CH, D), jnp.float32),
            pltpu.SemaphoreType.DMA((NBUF,)),
            pltpu.SemaphoreType.DMA((NBUF,)),
            pltpu.VMEM((rps, D), jnp.float32),
            pltpu.VMEM((rps, D), jnp.float32),
            pltpu.VMEM((rps, D), jnp.float32),
            pltpu.VMEM((rps, D), jnp.float32),
            pltpu.VMEM((rps, D), jnp.float32),
            pltpu.VMEM((rps, D), jnp.float32),
            pltpu.VMEM((D,), jnp.float32),
            pltpu.VMEM_SHARED((npad, D), jnp.float32),
            pltpu.VMEM_SHARED((npad, D), jnp.float32),
        ],
        compiler_params=pltpu.CompilerParams(use_tc_tiling_on_sc=False),
    )
    return f(s1p, self1, dis, b1, src, dst, zeros)


def _deg_body(n_chunks, rows_per_sub, dst_hbm, ones_hbm, zeros_hbm,
              out_hbm, dst_v, ones_v, sem, acc):
    c = lax.axis_index("c")
    s = lax.axis_index("s")
    wid = s * 2 + c
    sl = pl.ds(s * rows_per_sub, rows_per_sub)
    pltpu.sync_copy(zeros_hbm.at[sl], acc.at[sl])
    pltpu.sync_copy(ones_hbm, ones_v)
    pltpu.sync_copy(dst_hbm.at[wid], dst_v)
    plsc.subcore_barrier()

    n_outer = n_chunks // NBUF

    def body(o, carry):
        for b in range(NBUF):
            pltpu.async_copy(ones_v, acc.at[dst_v.at[o * NBUF + b]], sem,
                             add=True)
        for b in range(NBUF):
            pltpu.make_async_copy(ones_v, acc.at[pl.ds(0, CH)], sem).wait()
        return carry

    lax.fori_loop(0, n_outer, body, 0)
    plsc.subcore_barrier()
    pltpu.sync_copy(acc.at[sl], out_hbm.at[c, sl])


def _seg_sum_call(n_chunks, npad, tbl, src, dst, zeros):
    rps = npad // 16
    f = pl.kernel(
        functools.partial(_seg_sum_body, n_chunks, rps),
        out_type=jax.ShapeDtypeStruct((2, npad, D), jnp.float32),
        mesh=_mesh(),
        scratch_types=[
            pltpu.VMEM((n_chunks, CH), jnp.int32),
            pltpu.VMEM((n_chunks, CH), jnp.int32),
            pltpu.VMEM((NBUF, CH, D), jnp.float32),
            pltpu.SemaphoreType.DMA((NBUF,)),
            pltpu.SemaphoreType.DMA((NBUF,)),
            pltpu.VMEM_SHARED((npad, D), jnp.float32),
            pltpu.VMEM_SHARED((npad, D), jnp.float32),
        ],
        compiler_params=pltpu.CompilerParams(use_tc_tiling_on_sc=False),
    )
    return f(tbl, src, dst, zeros)


def _deg_call(n_chunks, npad, dst, ones1, zeros1):
    rps = npad // 16
    f = pl.kernel(
        functools.partial(_deg_body, n_chunks, rps),
        out_type=jax.ShapeDtypeStruct((2, npad), jnp.float32),
        mesh=_mesh(),
        scratch_types=[
            pltpu.VMEM((n_chunks, CH), jnp.int32),
            pltpu.VMEM((CH,), jnp.float32),
            pltpu.SemaphoreType.DMA,
            pltpu.VMEM_SHARED((npad,), jnp.float32),
        ],
        compiler_params=pltpu.CompilerParams(use_tc_tiling_on_sc=False),
    )
    return f(dst, ones1, zeros1)


def _tc_b_body(x_ref, w1_ref, degp_ref, g1_ref, self1_ref, dis_ref):
    deg = degp_ref[0] + degp_ref[1] + 1.0
    dis = lax.rsqrt(deg)
    h1 = jnp.dot(x_ref[...], w1_ref[...], preferred_element_type=jnp.float32)
    g1_ref[...] = h1 * dis
    self1_ref[...] = h1 / deg
    dis_ref[...] = jnp.broadcast_to(dis, h1.shape)


def _tc_f_body(s2p_ref, self2_ref, dis_ref, w2_ref, b2_ref, out_ref):
    a2 = dis_ref[...] * (s2p_ref[0] + s2p_ref[1]) + self2_ref[...]
    out_ref[...] = (
        jnp.dot(a2, w2_ref[...], preferred_element_type=jnp.float32) + b2_ref[...]
    )


def kernel(x, edge_index, W1, b1, W2, b2):
    n, d_in = x.shape
    e = edge_index.shape[1]
    d_out = W2.shape[1]
    n_chunks = -(-e // (NW * CH * NBUF)) * NBUF
    epad = NW * n_chunks * CH
    npad = -(-(n + 1) // 128) * 128

    idt = edge_index.dtype
    pad_idx = jnp.full((epad - e,), n, dtype=idt)
    src = jnp.concatenate([edge_index[0], pad_idx]).reshape(NW, n_chunks, CH)
    dst = jnp.concatenate([edge_index[1], pad_idx]).reshape(NW, n_chunks, CH)
    zeros = jnp.zeros((npad, D), jnp.float32)
    ones1 = jnp.ones((CH,), jnp.float32)
    zeros1 = jnp.zeros((npad,), jnp.float32)

    degp = _deg_call(n_chunks, npad, dst, ones1, zeros1)

    blk = n
    grid = 1
    g1, self1, dis = pl.pallas_call(
        _tc_b_body,
        grid=(grid,),
        in_specs=[
            pl.BlockSpec((blk, d_in), lambda i: (i, 0)),
            pl.BlockSpec((d_in, D), lambda i: (0, 0)),
            pl.BlockSpec((2, blk, 1), lambda i: (0, i, 0)),
        ],
        out_specs=[pl.BlockSpec((blk, D), lambda i: (i, 0))] * 3,
        out_shape=[jax.ShapeDtypeStruct((npad, D), jnp.float32)] * 3,
    )(x, W1, degp[:, :, None])

    s1p = _seg_sum_call(n_chunks, npad, g1, src, dst, zeros)

    s2p, self2 = _seg_sum2_call(n_chunks, npad, s1p, self1, dis, b1, src,
                                dst, zeros)

    out = pl.pallas_call(
        _tc_f_body,
        grid=(grid,),
        in_specs=[
            pl.BlockSpec((2, blk, D), lambda i: (0, i, 0)),
            pl.BlockSpec((blk, D), lambda i: (i, 0)),
            pl.BlockSpec((blk, D), lambda i: (i, 0)),
            pl.BlockSpec((D, d_out), lambda i: (0, 0)),
            pl.BlockSpec((1, d_out), lambda i: (0, 0)),
        ],
        out_specs=pl.BlockSpec((blk, d_out), lambda i: (i, 0)),
        out_shape=jax.ShapeDtypeStruct((n, d_out), jnp.float32),
    )(s2p, self2, dis, W2, b2.reshape(1, d_out))

    return out


# speedup vs baseline: 1.0852x; 1.0820x over previous
import functools

import jax
import jax.numpy as jnp
from jax import lax
from jax.experimental import pallas as pl
from jax.experimental.pallas import tpu as pltpu
from jax.experimental.pallas import tpu_sc as plsc

D = 16
NW = 32
CH = 128


def _mesh():
    return plsc.VectorSubcoreMesh(core_axis_name="c", subcore_axis_name="s")


NBUF = 8


def _load_idx(per_tile, t_full, rem, sent_rows, wid, idx_hbm, sent_hbm, idx_v):
    if rem == 0:
        pltpu.sync_copy(idx_hbm.at[pl.ds(wid * per_tile, per_tile)], idx_v)
        return

    @pl.when(wid < t_full)
    def _full():
        pltpu.sync_copy(idx_hbm.at[pl.ds(wid * per_tile, per_tile)], idx_v)

    @pl.when(wid == t_full)
    def _last():
        pltpu.sync_copy(idx_hbm.at[pl.ds(t_full * per_tile, rem)],
                        idx_v.at[pl.ds(0, rem)])
        pltpu.sync_copy(sent_hbm, idx_v.at[pl.ds(rem, sent_rows)])


def _tile_outer(per_tile, t_full, rem, sent_rows, wid):
    if rem == 0:
        return per_tile // NBUF
    last = rem + sent_rows
    return jnp.where(wid == t_full, last // NBUF, per_tile // NBUF)


def _seg_sum_body(geo, rows_per_sub, tbl_hbm, src_hbm, dst_hbm, sent_hbm,
                  zeros_hbm, out_hbm, src_v, dst_v, bufs, gsem, ssem, acc,
                  tbl_s):
    per_tile, t_full, rem, sent_rows = geo
    c = lax.axis_index("c")
    s = lax.axis_index("s")
    wid = s * 2 + c
    sl = pl.ds(s * rows_per_sub, rows_per_sub)
    pltpu.sync_copy(zeros_hbm.at[sl], acc.at[sl])
    pltpu.sync_copy(tbl_hbm.at[sl], tbl_s.at[sl])
    _load_idx(per_tile, t_full, rem, sent_rows, wid, src_hbm, sent_hbm, src_v)
    _load_idx(per_tile, t_full, rem, sent_rows, wid, dst_hbm, sent_hbm, dst_v)
    plsc.subcore_barrier()

    def gather_start(j, b):
        pltpu.async_copy(tbl_s.at[src_v.at[j]], bufs.at[b], gsem.at[b])

    def gather_wait(b):
        pltpu.make_async_copy(zeros_hbm.at[pl.ds(0, CH)], bufs.at[b],
                              gsem.at[b]).wait()

    def scatter_start(j, b):
        pltpu.async_copy(bufs.at[b], acc.at[dst_v.at[j]], ssem.at[b], add=True)

    def scatter_wait(b):
        pltpu.make_async_copy(bufs.at[b], acc.at[pl.ds(0, CH)],
                              ssem.at[b]).wait()

    for b in range(NBUF):
        gather_start(b, b)

    n_outer = _tile_outer(per_tile, t_full, rem, sent_rows, wid)

    def outer(o, carry):
        for b in range(NBUF):
            gather_wait(b)
            scatter_start(o * NBUF + b, b)

        @pl.when(o < n_outer - 1)
        def _prefetch():
            for b in range(NBUF):
                scatter_wait(b)
                gather_start((o + 1) * NBUF + b, b)

        return carry

    lax.fori_loop(0, n_outer, outer, 0)
    for b in range(NBUF):
        scatter_wait(b)
    plsc.subcore_barrier()
    pltpu.sync_copy(acc.at[sl], out_hbm.at[c, sl])


def _seg_sum2_body(geo, rows_per_sub, s1p_hbm, self1_hbm, dis_hbm,
                   b1_hbm, src_hbm, dst_hbm, sent_hbm, zeros_hbm,
                   out_hbm, self2_hbm, src_v, dst_v, bufs, gsem, ssem,
                   p0_v, p1_v, self1_v, dis_v, g2_v, self2_v, b1_v,
                   acc, tbl_s):
    per_tile, t_full, rem, sent_rows = geo
    c = lax.axis_index("c")
    s = lax.axis_index("s")
    wid = s * 2 + c
    base = s * rows_per_sub
    sl = pl.ds(base, rows_per_sub)
    pltpu.sync_copy(zeros_hbm.at[sl], acc.at[sl])
    pltpu.sync_copy(s1p_hbm.at[0, sl], p0_v)
    pltpu.sync_copy(s1p_hbm.at[1, sl], p1_v)
    pltpu.sync_copy(self1_hbm.at[sl], self1_v)
    pltpu.sync_copy(dis_hbm.at[sl], dis_v)
    pltpu.sync_copy(b1_hbm, b1_v)
    _load_idx(per_tile, t_full, rem, sent_rows, wid, src_hbm, sent_hbm, src_v)
    _load_idx(per_tile, t_full, rem, sent_rows, wid, dst_hbm, sent_hbm, dst_v)
    b1 = b1_v[...]

    def row(r, carry):
        dis = dis_v[r]
        h = jnp.maximum(dis * (p0_v[r] + p1_v[r]) + self1_v[r] + b1, 0.0)
        g2_v[r] = h * dis
        self2_v[r] = h * dis * dis
        return carry

    lax.fori_loop(0, rows_per_sub, row, 0)
    pltpu.sync_copy(g2_v, tbl_s.at[sl])

    @pl.when(c == 0)
    def _write_self2():
        pltpu.sync_copy(self2_v, self2_hbm.at[sl])

    plsc.subcore_barrier()

    def gather_start(j, b):
        pltpu.async_copy(tbl_s.at[src_v.at[j]], bufs.at[b], gsem.at[b])

    def gather_wait(b):
        pltpu.make_async_copy(zeros_hbm.at[pl.ds(0, CH)], bufs.at[b],
                              gsem.at[b]).wait()

    def scatter_start(j, b):
        pltpu.async_copy(bufs.at[b], acc.at[dst_v.at[j]], ssem.at[b], add=True)

    def scatter_wait(b):
        pltpu.make_async_copy(bufs.at[b], acc.at[pl.ds(0, CH)],
                              ssem.at[b]).wait()

    for b in range(NBUF):
        gather_start(b, b)

    n_outer = _tile_outer(per_tile, t_full, rem, sent_rows, wid)

    def outer(o, carry):
        for b in range(NBUF):
            gather_wait(b)
            scatter_start(o * NBUF + b, b)

        @pl.when(o < n_outer - 1)
        def _prefetch():
            for b in range(NBUF):
                scatter_wait(b)
                gather_start((o + 1) * NBUF + b, b)

        return carry

    lax.fori_loop(0, n_outer, outer, 0)
    for b in range(NBUF):
        scatter_wait(b)
    plsc.subcore_barrier()
    pltpu.sync_copy(acc.at[sl], out_hbm.at[c, sl])


def _seg_sum2_call(geo, npad, s1p, self1, dis, b1, src, dst, sent, zeros):
    rps = npad // 16
    f = pl.kernel(
        functools.partial(_seg_sum2_body, geo, rps),
        out_type=(
            jax.ShapeDtypeStruct((2, npad, D), jnp.float32),
            jax.ShapeDtypeStruct((npad, D), jnp.float32),
        ),
        mesh=_mesh(),
        scratch_types=[
            pltpu.VMEM((geo[0], CH), jnp.int32),
            pltpu.VMEM((geo[0], CH), jnp.int32),
            pltpu.VMEM((NBUF, CH, D), jnp.float32),
            pltpu.SemaphoreType.DMA((NBUF,)),
            pltpu.SemaphoreType.DMA((NBUF,)),
            pltpu.VMEM((rps, D), jnp.float32),
            pltpu.VMEM((rps, D), jnp.float32),
            pltpu.VMEM((rps, D), jnp.float32),
            pltpu.VMEM((rps, D), jnp.float32),
            pltpu.VMEM((rps, D), jnp.float32),
            pltpu.VMEM((rps, D), jnp.float32),
            pltpu.VMEM((D,), jnp.float32),
            pltpu.VMEM_SHARED((npad, D), jnp.float32),
            pltpu.VMEM_SHARED((npad, D), jnp.float32),
        ],
        compiler_params=pltpu.CompilerParams(use_tc_tiling_on_sc=False),
    )
    return f(s1p, self1, dis, b1, src, dst, sent, zeros)


def _deg_body(geo, rows_per_sub, dst_hbm, sent_hbm, ones_hbm, zeros_hbm,
              out_hbm, dst_v, ones_v, sem, acc):
    per_tile, t_full, rem, sent_rows = geo
    c = lax.axis_index("c")
    s = lax.axis_index("s")
    wid = s * 2 + c
    sl = pl.ds(s * rows_per_sub, rows_per_sub)
    pltpu.sync_copy(zeros_hbm.at[sl], acc.at[sl])
    pltpu.sync_copy(ones_hbm, ones_v)
    _load_idx(per_tile, t_full, rem, sent_rows, wid, dst_hbm, sent_hbm, dst_v)
    plsc.subcore_barrier()

    n_outer = _tile_outer(per_tile, t_full, rem, sent_rows, wid)

    def body(o, carry):
        for b in range(NBUF):
            pltpu.async_copy(ones_v, acc.at[dst_v.at[o * NBUF + b]], sem,
                             add=True)
        for b in range(NBUF):
            pltpu.make_async_copy(ones_v, acc.at[pl.ds(0, CH)], sem).wait()
        return carry

    lax.fori_loop(0, n_outer, body, 0)
    plsc.subcore_barrier()
    pltpu.sync_copy(acc.at[sl], out_hbm.at[c, sl])


def _seg_sum_call(geo, npad, tbl, src, dst, sent, zeros):
    rps = npad // 16
    f = pl.kernel(
        functools.partial(_seg_sum_body, geo, rps),
        out_type=jax.ShapeDtypeStruct((2, npad, D), jnp.float32),
        mesh=_mesh(),
        scratch_types=[
            pltpu.VMEM((geo[0], CH), jnp.int32),
            pltpu.VMEM((geo[0], CH), jnp.int32),
            pltpu.VMEM((NBUF, CH, D), jnp.float32),
            pltpu.SemaphoreType.DMA((NBUF,)),
            pltpu.SemaphoreType.DMA((NBUF,)),
            pltpu.VMEM_SHARED((npad, D), jnp.float32),
            pltpu.VMEM_SHARED((npad, D), jnp.float32),
        ],
        compiler_params=pltpu.CompilerParams(use_tc_tiling_on_sc=False),
    )
    return f(tbl, src, dst, sent, zeros)


def _deg_call(geo, npad, dst, sent, ones1, zeros1):
    rps = npad // 16
    f = pl.kernel(
        functools.partial(_deg_body, geo, rps),
        out_type=jax.ShapeDtypeStruct((2, npad), jnp.float32),
        mesh=_mesh(),
        scratch_types=[
            pltpu.VMEM((geo[0], CH), jnp.int32),
            pltpu.VMEM((CH,), jnp.float32),
            pltpu.SemaphoreType.DMA,
            pltpu.VMEM_SHARED((npad,), jnp.float32),
        ],
        compiler_params=pltpu.CompilerParams(use_tc_tiling_on_sc=False),
    )
    return f(dst, sent, ones1, zeros1)


def _tc_b_body(x_ref, w1_ref, degp_ref, g1_ref, self1_ref, dis_ref):
    deg = degp_ref[0] + degp_ref[1] + 1.0
    dis = lax.rsqrt(deg)
    h1 = jnp.dot(x_ref[...], w1_ref[...], preferred_element_type=jnp.float32)
    g1_ref[...] = h1 * dis
    self1_ref[...] = h1 / deg
    dis_ref[...] = jnp.broadcast_to(dis, h1.shape)


def _tc_f_body(s2p_ref, self2_ref, dis_ref, w2_ref, b2_ref, out_ref):
    a2 = dis_ref[...] * (s2p_ref[0] + s2p_ref[1]) + self2_ref[...]
    out_ref[...] = (
        jnp.dot(a2, w2_ref[...], preferred_element_type=jnp.float32) + b2_ref[...]
    )


def kernel(x, edge_index, W1, b1, W2, b2):
    n, d_in = x.shape
    e = edge_index.shape[1]
    d_out = W2.shape[1]
    n_rows = e // CH
    per_tile = -(-(-(-n_rows // NW)) // NBUF) * NBUF
    t_full = n_rows // per_tile
    rem = n_rows - t_full * per_tile
    sent_rows = (-(-rem // NBUF) * NBUF - rem) if rem else 0
    geo = (per_tile, t_full, rem, sent_rows)
    npad = -(-(n + 1) // 128) * 128

    idt = edge_index.dtype
    src = edge_index[0].reshape(n_rows, CH)
    dst = edge_index[1].reshape(n_rows, CH)
    sent = jnp.full((max(sent_rows, 1), CH), n, dtype=idt)
    zeros = jnp.zeros((npad, D), jnp.float32)
    ones1 = jnp.ones((CH,), jnp.float32)
    zeros1 = jnp.zeros((npad,), jnp.float32)

    degp = _deg_call(geo, npad, dst, sent, ones1, zeros1)

    blk = n
    grid = 1
    g1, self1, dis = pl.pallas_call(
        _tc_b_body,
        grid=(grid,),
        in_specs=[
            pl.BlockSpec((blk, d_in), lambda i: (i, 0)),
            pl.BlockSpec((d_in, D), lambda i: (0, 0)),
            pl.BlockSpec((2, blk, 1), lambda i: (0, i, 0)),
        ],
        out_specs=[pl.BlockSpec((blk, D), lambda i: (i, 0))] * 3,
        out_shape=[jax.ShapeDtypeStruct((npad, D), jnp.float32)] * 3,
    )(x, W1, degp[:, :, None])

    s1p = _seg_sum_call(geo, npad, g1, src, dst, sent, zeros)

    s2p, self2 = _seg_sum2_call(geo, npad, s1p, self1, dis, b1, src,
                                dst, sent, zeros)

    out = pl.pallas_call(
        _tc_f_body,
        grid=(grid,),
        in_specs=[
            pl.BlockSpec((2, blk, D), lambda i: (0, i, 0)),
            pl.BlockSpec((blk, D), lambda i: (i, 0)),
            pl.BlockSpec((blk, D), lambda i: (i, 0)),
            pl.BlockSpec((D, d_out), lambda i: (0, 0)),
            pl.BlockSpec((1, d_out), lambda i: (0, 0)),
        ],
        out_specs=pl.BlockSpec((blk, d_out), lambda i: (i, 0)),
        out_shape=jax.ShapeDtypeStruct((n, d_out), jnp.float32),
    )(s2p, self2, dis, W2, b2.reshape(1, d_out))

    return out
